# Initial kernel scaffold; baseline (speedup 1.0000x reference)
import jax
import jax.numpy as jnp
from jax.experimental import pallas as pl


def kernel(node_feat, memory, time_w, time_b, Wq, Wk, Wv, Wm1, bm1, Wm2, bm2, source_nodes, destination_nodes, negative_nodes, edge_times, ngh_idx, ngh_times):
    raise NotImplementedError("write your pallas kernel here")


# speedup vs baseline: 2.7517x; 2.7517x over previous
import jax
import jax.numpy as jnp
from jax import lax
from jax.experimental import pallas as pl
from jax.experimental.pallas import tpu as pltpu
from jax.experimental.pallas import tpu_sc as plsc

N = 100000
D = 128
B = 16384
TB = 3 * B
K = 20
H = 2
DH = D // H


_CRB = 1000


def _combine_body(nf_ref, mem_ref, out_ref):
    out_ref[...] = nf_ref[...] + mem_ref[...]


def _combine(node_feat, memory):
    return pl.pallas_call(
        _combine_body,
        grid=(N // _CRB,),
        in_specs=[pl.BlockSpec((_CRB, D), lambda i: (i, 0)),
                  pl.BlockSpec((_CRB, D), lambda i: (i, 0))],
        out_specs=pl.BlockSpec((_CRB, D), lambda i: (i, 0)),
        out_shape=jax.ShapeDtypeStruct((N, D), jnp.float32),
    )(node_feat, memory)


_NC, _NS = 2, 16
_NW = _NC * _NS
_GN = TB * K
_PWN = _GN // _NW
_PWH = TB // _NW
_CH = 128


def _gather_body(table, idx_n, idx_h, out_n, out_h, idx_v, rows_v, sem):
    wid = lax.axis_index("s") * _NC + lax.axis_index("c")

    def _chunked(idx_hbm, out_hbm, base, n_chunks):
        def step(i, carry):
            off = base + i * _CH
            pltpu.sync_copy(idx_hbm.at[pl.ds(off, _CH)], idx_v)
            pltpu.async_copy(table.at[idx_v], rows_v, sem).wait()
            pltpu.sync_copy(rows_v, out_hbm.at[pl.ds(off, _CH)])
            return carry
        lax.fori_loop(0, n_chunks, step, 0)

    _chunked(idx_n, out_n, wid * _PWN, _PWN // _CH)
    _chunked(idx_h, out_h, wid * _PWH, _PWH // _CH)


def _sc_gather(table, idx_n, idx_h):
    kfn = pl.kernel(
        _gather_body,
        out_type=(jax.ShapeDtypeStruct((_GN, D), jnp.float32),
                  jax.ShapeDtypeStruct((TB, D), jnp.float32)),
        mesh=plsc.VectorSubcoreMesh(core_axis_name="c", subcore_axis_name="s"),
        scratch_types=[
            pltpu.VMEM((_CH,), jnp.int32),
            pltpu.VMEM((_CH, D), jnp.float32),
            pltpu.SemaphoreType.DMA,
        ],
    )
    return kfn(table, idx_n, idx_h)


_R = 128


def _attn_body(h_ref, ngh_ref, nt_ref, ts_ref, tw_ref, tb_ref, wq_ref,
               wk_ref, wv_ref, wm1_ref, bm1_ref, wm2_ref, bm2_ref, out_ref):
    h = h_ref[...]
    ngh2 = ngh_ref[...].reshape(K * _R, D)
    nt = nt_ref[...]
    ts = ts_ref[...]
    w = tw_ref[...]
    b = tb_ref[...]

    dt = ts - nt
    te = jnp.cos(dt[:, :, None] * w[None, :, :] + b[None, :, :])
    te2 = te.reshape(K * _R, D)

    wk = wk_ref[...]
    wv = wv_ref[...]
    kmat = ngh2 @ wk[:D] + te2 @ wk[D:]
    vmat = ngh2 @ wv[:D] + te2 @ wv[D:]

    wq = wq_ref[...]
    q = h @ wq[:D] + jnp.cos(b) @ wq[D:]

    k3 = kmat.reshape(K, _R, D)
    v3 = vmat.reshape(K, _R, D)
    scale = 1.0 / (DH ** 0.5)
    s1 = jnp.sum(k3[:, :, :DH] * q[None, :, :DH], axis=-1) * scale
    s2 = jnp.sum(k3[:, :, DH:] * q[None, :, DH:], axis=-1) * scale
    a1 = jax.nn.softmax(s1, axis=0)
    a2 = jax.nn.softmax(s2, axis=0)
    o1 = jnp.sum(v3[:, :, :DH] * a1[:, :, None], axis=0)
    o2 = jnp.sum(v3[:, :, DH:] * a2[:, :, None], axis=0)

    wm1 = wm1_ref[...]
    out_cat = jnp.concatenate([o1, o2], axis=-1)
    pre = out_cat @ wm1[:D] + h @ wm1[D:] + bm1_ref[...]
    out_ref[...] = jnp.maximum(pre, 0.0) @ wm2_ref[...] + bm2_ref[...]


def _attn_call(h, ngh3, nt_t, ts2, tw2, tb2, Wq, Wk, Wv, Wm1, bm1_2, Wm2, bm2_2):
    full = lambda shape: pl.BlockSpec(shape, lambda i: tuple(0 for _ in shape))
    return pl.pallas_call(
        _attn_body,
        grid=(TB // _R,),
        in_specs=[
            pl.BlockSpec((_R, D), lambda i: (i, 0)),
            pl.BlockSpec((K, _R, D), lambda i: (0, i, 0)),
            pl.BlockSpec((K, _R), lambda i: (0, i)),
            pl.BlockSpec((1, _R), lambda i: (0, i)),
            full((1, D)), full((1, D)),
            full((2 * D, D)), full((2 * D, D)), full((2 * D, D)),
            full((2 * D, D)), full((1, D)),
            full((D, D)), full((1, D)),
        ],
        out_specs=pl.BlockSpec((_R, D), lambda i: (i, 0)),
        out_shape=jax.ShapeDtypeStruct((TB, D), jnp.float32),
    )(h, ngh3, nt_t, ts2, tw2, tb2, Wq, Wk, Wv, Wm1, bm1_2, Wm2, bm2_2)


def kernel(node_feat, memory, time_w, time_b, Wq, Wk, Wv, Wm1, bm1, Wm2, bm2,
           source_nodes, destination_nodes, negative_nodes, edge_times,
           ngh_idx, ngh_times):
    nodes = jnp.concatenate(
        [source_nodes, destination_nodes, negative_nodes]).astype(jnp.int32)
    ts3 = jnp.concatenate([edge_times, edge_times, edge_times])

    combined = _combine(node_feat, memory)

    idx_n = ngh_idx.astype(jnp.int32).T.reshape(-1)
    ngh_flat, h = _sc_gather(combined, idx_n, nodes)
    ngh3 = ngh_flat.reshape(K, TB, D)

    emb = _attn_call(
        h, ngh3, ngh_times.T, ts3.reshape(1, TB),
        time_w.reshape(1, D), time_b.reshape(1, D),
        Wq, Wk, Wv, Wm1, bm1.reshape(1, D), Wm2, bm2.reshape(1, D))
    return emb


# speedup vs baseline: 4.3777x; 1.5909x over previous
import jax
import jax.numpy as jnp
from jax import lax
from jax.experimental import pallas as pl
from jax.experimental.pallas import tpu as pltpu
from jax.experimental.pallas import tpu_sc as plsc

N = 100000
D = 128
B = 16384
TB = 3 * B
K = 20
H = 2
DH = D // H


_CRB = 1000


def _combine_body(nf_ref, mem_ref, out_ref):
    out_ref[...] = nf_ref[...] + mem_ref[...]


def _combine(node_feat, memory):
    return pl.pallas_call(
        _combine_body,
        grid=(N // _CRB,),
        in_specs=[pl.BlockSpec((_CRB, D), lambda i: (i, 0)),
                  pl.BlockSpec((_CRB, D), lambda i: (i, 0))],
        out_specs=pl.BlockSpec((_CRB, D), lambda i: (i, 0)),
        out_shape=jax.ShapeDtypeStruct((N, D), jnp.float32),
    )(node_feat, memory)


_NC, _NS = 2, 16
_NW = _NC * _NS
_GN = TB * K
_PWN = _GN // _NW
_PWH = TB // _NW
_CH = 128


def _gather_body(table, idx_n, idx_h, out_n, out_h, idx_v, rows_v, sem):
    wid = lax.axis_index("s") * _NC + lax.axis_index("c")

    def _chunked(idx_hbm, out_hbm, base, n_chunks):
        def step(i, carry):
            off = base + i * _CH
            pltpu.sync_copy(idx_hbm.at[pl.ds(off, _CH)], idx_v)
            pltpu.async_copy(table.at[idx_v], rows_v, sem).wait()
            pltpu.sync_copy(rows_v, out_hbm.at[pl.ds(off, _CH)])
            return carry
        lax.fori_loop(0, n_chunks, step, 0)

    _chunked(idx_n, out_n, wid * _PWN, _PWN // _CH)
    _chunked(idx_h, out_h, wid * _PWH, _PWH // _CH)


def _sc_gather(table, idx_n, idx_h):
    kfn = pl.kernel(
        _gather_body,
        out_type=(jax.ShapeDtypeStruct((_GN, D), jnp.float32),
                  jax.ShapeDtypeStruct((TB, D), jnp.float32)),
        mesh=plsc.VectorSubcoreMesh(core_axis_name="c", subcore_axis_name="s"),
        scratch_types=[
            pltpu.VMEM((_CH,), jnp.int32),
            pltpu.VMEM((_CH, D), jnp.float32),
            pltpu.SemaphoreType.DMA,
        ],
    )
    return kfn(table, idx_n, idx_h)


_R = 128

_INV2PI = 0.15915494309189535
_COSC = (9.9999921088e-01, -1.9738980362e+01, 6.4928657530e+01,
         -8.5271622212e+01, 5.8790493573e+01, -2.1071105911e+01)


def _fast_cos(x):
    r = x * _INV2PI
    f = r - jnp.floor(r + 0.5)
    u = f * f
    p = jnp.float32(_COSC[5])
    for c in (_COSC[4], _COSC[3], _COSC[2], _COSC[1], _COSC[0]):
        p = p * u + jnp.float32(c)
    return p


def _attn_body(h_ref, ngh_ref, nt_ref, ts_ref, tw_ref, tb_ref, wq_ref,
               wk_ref, wv_ref, wm1_ref, bm1_ref, wm2_ref, bm2_ref, out_ref):
    h = h_ref[...]
    ngh2 = ngh_ref[...].reshape(K * _R, D)
    nt = nt_ref[...]
    ts = ts_ref[...]
    w = tw_ref[...]
    b = tb_ref[...]

    dt = ts - nt
    te = _fast_cos(dt[:, :, None] * w[None, :, :] + b[None, :, :])
    te2 = te.reshape(K * _R, D)

    wk = wk_ref[...]
    wv = wv_ref[...]
    kmat = ngh2 @ wk[:D] + te2 @ wk[D:]
    vmat = ngh2 @ wv[:D] + te2 @ wv[D:]

    wq = wq_ref[...]
    q = h @ wq[:D] + jnp.cos(b) @ wq[D:]

    k3 = kmat.reshape(K, _R, D)
    v3 = vmat.reshape(K, _R, D)
    scale = 1.0 / (DH ** 0.5)
    s1 = jnp.sum(k3[:, :, :DH] * q[None, :, :DH], axis=-1) * scale
    s2 = jnp.sum(k3[:, :, DH:] * q[None, :, DH:], axis=-1) * scale
    a1 = jax.nn.softmax(s1, axis=0)
    a2 = jax.nn.softmax(s2, axis=0)
    o1 = jnp.sum(v3[:, :, :DH] * a1[:, :, None], axis=0)
    o2 = jnp.sum(v3[:, :, DH:] * a2[:, :, None], axis=0)

    wm1 = wm1_ref[...]
    out_cat = jnp.concatenate([o1, o2], axis=-1)
    pre = out_cat @ wm1[:D] + h @ wm1[D:] + bm1_ref[...]
    out_ref[...] = jnp.maximum(pre, 0.0) @ wm2_ref[...] + bm2_ref[...]


def _attn_call(h, ngh3, nt_t, ts2, tw2, tb2, Wq, Wk, Wv, Wm1, bm1_2, Wm2, bm2_2):
    full = lambda shape: pl.BlockSpec(shape, lambda i: tuple(0 for _ in shape))
    return pl.pallas_call(
        _attn_body,
        grid=(TB // _R,),
        in_specs=[
            pl.BlockSpec((_R, D), lambda i: (i, 0)),
            pl.BlockSpec((K, _R, D), lambda i: (0, i, 0)),
            pl.BlockSpec((K, _R), lambda i: (0, i)),
            pl.BlockSpec((1, _R), lambda i: (0, i)),
            full((1, D)), full((1, D)),
            full((2 * D, D)), full((2 * D, D)), full((2 * D, D)),
            full((2 * D, D)), full((1, D)),
            full((D, D)), full((1, D)),
        ],
        out_specs=pl.BlockSpec((_R, D), lambda i: (i, 0)),
        out_shape=jax.ShapeDtypeStruct((TB, D), jnp.float32),
    )(h, ngh3, nt_t, ts2, tw2, tb2, Wq, Wk, Wv, Wm1, bm1_2, Wm2, bm2_2)


def kernel(node_feat, memory, time_w, time_b, Wq, Wk, Wv, Wm1, bm1, Wm2, bm2,
           source_nodes, destination_nodes, negative_nodes, edge_times,
           ngh_idx, ngh_times):
    nodes = jnp.concatenate(
        [source_nodes, destination_nodes, negative_nodes]).astype(jnp.int32)
    ts3 = jnp.concatenate([edge_times, edge_times, edge_times])

    combined = _combine(node_feat, memory)

    idx_n = ngh_idx.astype(jnp.int32).T.reshape(-1)
    ngh_flat, h = _sc_gather(combined, idx_n, nodes)
    ngh3 = ngh_flat.reshape(K, TB, D)

    emb = _attn_call(
        h, ngh3, ngh_times.T, ts3.reshape(1, TB),
        time_w.reshape(1, D), time_b.reshape(1, D),
        Wq, Wk, Wv, Wm1, bm1.reshape(1, D), Wm2, bm2.reshape(1, D))
    return emb


# speedup vs baseline: 4.9827x; 1.1382x over previous
import jax
import jax.numpy as jnp
from jax import lax
from jax.experimental import pallas as pl
from jax.experimental.pallas import tpu as pltpu
from jax.experimental.pallas import tpu_sc as plsc

N = 100000
D = 128
B = 16384
TB = 3 * B
K = 20
H = 2
DH = D // H


_CRB = 1000


def _combine_body(nf_ref, mem_ref, out_ref):
    out_ref[...] = nf_ref[...] + mem_ref[...]


def _combine(node_feat, memory):
    return pl.pallas_call(
        _combine_body,
        grid=(N // _CRB,),
        in_specs=[pl.BlockSpec((_CRB, D), lambda i: (i, 0)),
                  pl.BlockSpec((_CRB, D), lambda i: (i, 0))],
        out_specs=pl.BlockSpec((_CRB, D), lambda i: (i, 0)),
        out_shape=jax.ShapeDtypeStruct((N, D), jnp.float32),
    )(node_feat, memory)


_NC, _NS = 2, 16
_NW = _NC * _NS
_GN = TB * K
_PWN = _GN // _NW
_PWH = TB // _NW
_CH = 128


def _gather_body(table, idx_n, idx_h, out_n, out_h, idxl, rows, sg0, sg1,
                 ss0, ss1):
    wid = lax.axis_index("s") * _NC + lax.axis_index("c")
    pltpu.sync_copy(idx_n.at[pl.ds(wid * _PWN, _PWN)], idxl.at[pl.ds(0, _PWN)])
    pltpu.sync_copy(idx_h.at[pl.ds(wid * _PWH, _PWH)],
                    idxl.at[pl.ds(_PWN, _PWH)])
    sg = (sg0, sg1)
    ss = (ss0, ss1)

    def run(ibase, out_hbm, obase, n_chunks):
        def g_copy(i, b):
            return pltpu.make_async_copy(
                table.at[idxl.at[pl.ds(ibase + i * _CH, _CH)]],
                rows.at[b], sg[b])

        def s_copy(i, b):
            return pltpu.make_async_copy(
                rows.at[b], out_hbm.at[pl.ds(obase + i * _CH, _CH)], ss[b])

        for b in (0, 1):
            g_copy(b, b).start()

        def body(g, carry):
            for b in (0, 1):
                i = 2 * g + b
                g_copy(i, b).wait()
                s_copy(i, b).start()
            for b in (0, 1):
                i = 2 * g + b

                def _prefetch(i=i, b=b):
                    s_copy(i, b).wait()
                    g_copy(i + 2, b).start()

                pl.when(i + 2 < n_chunks)(_prefetch)
            return carry

        lax.fori_loop(0, n_chunks // 2, body, 0)
        for b in (0, 1):
            s_copy(n_chunks - 2 + b, b).wait()

    run(0, out_n, wid * _PWN, _PWN // _CH)
    run(_PWN, out_h, wid * _PWH, _PWH // _CH)


def _sc_gather(table, idx_n, idx_h):
    kfn = pl.kernel(
        _gather_body,
        out_type=(jax.ShapeDtypeStruct((_GN, D), jnp.float32),
                  jax.ShapeDtypeStruct((TB, D), jnp.float32)),
        mesh=plsc.VectorSubcoreMesh(core_axis_name="c", subcore_axis_name="s"),
        scratch_types=[
            pltpu.VMEM((_PWN + _PWH,), jnp.int32),
            pltpu.VMEM((2, _CH, D), jnp.float32),
            pltpu.SemaphoreType.DMA,
            pltpu.SemaphoreType.DMA,
            pltpu.SemaphoreType.DMA,
            pltpu.SemaphoreType.DMA,
        ],
    )
    return kfn(table, idx_n, idx_h)


_R = 128

_INV2PI = 0.15915494309189535
_COSC = (9.9999921088e-01, -1.9738980362e+01, 6.4928657530e+01,
         -8.5271622212e+01, 5.8790493573e+01, -2.1071105911e+01)


def _fast_cos(x):
    r = x * _INV2PI
    f = r - jnp.floor(r + 0.5)
    u = f * f
    p = jnp.float32(_COSC[5])
    for c in (_COSC[4], _COSC[3], _COSC[2], _COSC[1], _COSC[0]):
        p = p * u + jnp.float32(c)
    return p


def _attn_body(h_ref, ngh_ref, nt_ref, ts_ref, tw_ref, tb_ref, wq_ref,
               wk_ref, wv_ref, wm1_ref, bm1_ref, wm2_ref, bm2_ref, out_ref):
    h = h_ref[...]
    ngh2 = ngh_ref[...].reshape(K * _R, D)
    nt = nt_ref[...]
    ts = ts_ref[...]
    w = tw_ref[...]
    b = tb_ref[...]

    dt = ts - nt
    te = _fast_cos(dt[:, :, None] * w[None, :, :] + b[None, :, :])
    te2 = te.reshape(K * _R, D)

    wk = wk_ref[...]
    wv = wv_ref[...]
    kmat = ngh2 @ wk[:D] + te2 @ wk[D:]
    vmat = ngh2 @ wv[:D] + te2 @ wv[D:]

    wq = wq_ref[...]
    q = h @ wq[:D] + jnp.cos(b) @ wq[D:]

    k3 = kmat.reshape(K, _R, D)
    v3 = vmat.reshape(K, _R, D)
    scale = 1.0 / (DH ** 0.5)
    s1 = jnp.sum(k3[:, :, :DH] * q[None, :, :DH], axis=-1) * scale
    s2 = jnp.sum(k3[:, :, DH:] * q[None, :, DH:], axis=-1) * scale
    a1 = jax.nn.softmax(s1, axis=0)
    a2 = jax.nn.softmax(s2, axis=0)
    o1 = jnp.sum(v3[:, :, :DH] * a1[:, :, None], axis=0)
    o2 = jnp.sum(v3[:, :, DH:] * a2[:, :, None], axis=0)

    wm1 = wm1_ref[...]
    out_cat = jnp.concatenate([o1, o2], axis=-1)
    pre = out_cat @ wm1[:D] + h @ wm1[D:] + bm1_ref[...]
    out_ref[...] = jnp.maximum(pre, 0.0) @ wm2_ref[...] + bm2_ref[...]


def _attn_call(h, ngh3, nt_t, ts2, tw2, tb2, Wq, Wk, Wv, Wm1, bm1_2, Wm2, bm2_2):
    full = lambda shape: pl.BlockSpec(shape, lambda i: tuple(0 for _ in shape))
    return pl.pallas_call(
        _attn_body,
        grid=(TB // _R,),
        in_specs=[
            pl.BlockSpec((_R, D), lambda i: (i, 0)),
            pl.BlockSpec((K, _R, D), lambda i: (0, i, 0)),
            pl.BlockSpec((K, _R), lambda i: (0, i)),
            pl.BlockSpec((1, _R), lambda i: (0, i)),
            full((1, D)), full((1, D)),
            full((2 * D, D)), full((2 * D, D)), full((2 * D, D)),
            full((2 * D, D)), full((1, D)),
            full((D, D)), full((1, D)),
        ],
        out_specs=pl.BlockSpec((_R, D), lambda i: (i, 0)),
        out_shape=jax.ShapeDtypeStruct((TB, D), jnp.float32),
    )(h, ngh3, nt_t, ts2, tw2, tb2, Wq, Wk, Wv, Wm1, bm1_2, Wm2, bm2_2)


def kernel(node_feat, memory, time_w, time_b, Wq, Wk, Wv, Wm1, bm1, Wm2, bm2,
           source_nodes, destination_nodes, negative_nodes, edge_times,
           ngh_idx, ngh_times):
    nodes = jnp.concatenate(
        [source_nodes, destination_nodes, negative_nodes]).astype(jnp.int32)
    ts3 = jnp.concatenate([edge_times, edge_times, edge_times])

    combined = _combine(node_feat, memory)

    idx_n = ngh_idx.astype(jnp.int32).T.reshape(-1)
    ngh_flat, h = _sc_gather(combined, idx_n, nodes)
    ngh3 = ngh_flat.reshape(K, TB, D)

    emb = _attn_call(
        h, ngh3, ngh_times.T, ts3.reshape(1, TB),
        time_w.reshape(1, D), time_b.reshape(1, D),
        Wq, Wk, Wv, Wm1, bm1.reshape(1, D), Wm2, bm2.reshape(1, D))
    return emb


# speedup vs baseline: 5.8615x; 1.1764x over previous
import functools

import jax
import jax.numpy as jnp
from jax import lax
from jax.experimental import pallas as pl
from jax.experimental.pallas import tpu as pltpu
from jax.experimental.pallas import tpu_sc as plsc

N = 100000
D = 128
B = 16384
TB = 3 * B
K = 20
H = 2
DH = D // H


_CRB = 1000


def _combine_body(nf_ref, mem_ref, out_ref):
    out_ref[...] = nf_ref[...] + mem_ref[...]


def _combine(node_feat, memory):
    return pl.pallas_call(
        _combine_body,
        grid=(N // _CRB,),
        in_specs=[pl.BlockSpec((_CRB, D), lambda i: (i, 0)),
                  pl.BlockSpec((_CRB, D), lambda i: (i, 0))],
        out_specs=pl.BlockSpec((_CRB, D), lambda i: (i, 0)),
        out_shape=jax.ShapeDtypeStruct((N, D), jnp.float32),
    )(node_feat, memory)


_NC, _NS = 2, 16
_NW = _NC * _NS
_CH = 128


@functools.lru_cache(maxsize=None)
def _make_sc_gather(tbc):
    pwn = tbc * K // _NW
    pwh = tbc // _NW
    assert pwn % (2 * _CH) == 0 and pwh % (2 * _CH) == 0

    def body(table, idx_n, idx_h, out_n, out_h, idxl, rows, sg0, sg1,
             ss0, ss1):
        wid = lax.axis_index("s") * _NC + lax.axis_index("c")
        pltpu.sync_copy(idx_n.at[pl.ds(wid * pwn, pwn)], idxl.at[pl.ds(0, pwn)])
        pltpu.sync_copy(idx_h.at[pl.ds(wid * pwh, pwh)],
                        idxl.at[pl.ds(pwn, pwh)])
        sg = (sg0, sg1)
        ss = (ss0, ss1)

        def run(ibase, out_hbm, obase, n_chunks):
            def g_copy(i, b):
                return pltpu.make_async_copy(
                    table.at[idxl.at[pl.ds(ibase + i * _CH, _CH)]],
                    rows.at[b], sg[b])

            def s_copy(i, b):
                return pltpu.make_async_copy(
                    rows.at[b], out_hbm.at[pl.ds(obase + i * _CH, _CH)], ss[b])

            for b in (0, 1):
                g_copy(b, b).start()

            def loop_body(g, carry):
                for b in (0, 1):
                    i = 2 * g + b
                    g_copy(i, b).wait()
                    s_copy(i, b).start()
                for b in (0, 1):
                    i = 2 * g + b

                    def _prefetch(i=i, b=b):
                        s_copy(i, b).wait()
                        g_copy(i + 2, b).start()

                    pl.when(i + 2 < n_chunks)(_prefetch)
                return carry

            lax.fori_loop(0, n_chunks // 2, loop_body, 0)
            for b in (0, 1):
                s_copy(n_chunks - 2 + b, b).wait()

        run(0, out_n, wid * pwn, pwn // _CH)
        run(pwn, out_h, wid * pwh, pwh // _CH)

    return pl.kernel(
        body,
        out_type=(jax.ShapeDtypeStruct((tbc * K, D), jnp.float32),
                  jax.ShapeDtypeStruct((tbc, D), jnp.float32)),
        mesh=plsc.VectorSubcoreMesh(core_axis_name="c", subcore_axis_name="s"),
        scratch_types=[
            pltpu.VMEM((pwn + pwh,), jnp.int32),
            pltpu.VMEM((2, _CH, D), jnp.float32),
            pltpu.SemaphoreType.DMA,
            pltpu.SemaphoreType.DMA,
            pltpu.SemaphoreType.DMA,
            pltpu.SemaphoreType.DMA,
        ],
    )


_R = 128

_INV2PI = 0.15915494309189535
_COSC = (9.9999921088e-01, -1.9738980362e+01, 6.4928657530e+01,
         -8.5271622212e+01, 5.8790493573e+01, -2.1071105911e+01)


def _fast_cos(x):
    r = x * _INV2PI
    f = r - jnp.floor(r + 0.5)
    u = f * f
    p = jnp.float32(_COSC[5])
    for c in (_COSC[4], _COSC[3], _COSC[2], _COSC[1], _COSC[0]):
        p = p * u + jnp.float32(c)
    return p


def _attn_body(h_ref, ngh_ref, nt_ref, ts_ref, tw_ref, tb_ref, wq_ref,
               wk_ref, wv_ref, wm1_ref, bm1_ref, wm2_ref, bm2_ref, out_ref):
    h = h_ref[...]
    ngh2 = ngh_ref[...].reshape(K * _R, D)
    nt = nt_ref[...]
    ts = ts_ref[...]
    w = tw_ref[...]
    b = tb_ref[...]

    dt = ts - nt
    te = _fast_cos(dt[:, :, None] * w[None, :, :] + b[None, :, :])
    te2 = te.reshape(K * _R, D)

    wk = wk_ref[...]
    wv = wv_ref[...]
    kmat = ngh2 @ wk[:D] + te2 @ wk[D:]
    vmat = ngh2 @ wv[:D] + te2 @ wv[D:]

    wq = wq_ref[...]
    q = h @ wq[:D] + jnp.cos(b) @ wq[D:]

    k3 = kmat.reshape(K, _R, D)
    v3 = vmat.reshape(K, _R, D)
    scale = 1.0 / (DH ** 0.5)
    s1 = jnp.sum(k3[:, :, :DH] * q[None, :, :DH], axis=-1) * scale
    s2 = jnp.sum(k3[:, :, DH:] * q[None, :, DH:], axis=-1) * scale
    a1 = jax.nn.softmax(s1, axis=0)
    a2 = jax.nn.softmax(s2, axis=0)
    o1 = jnp.sum(v3[:, :, :DH] * a1[:, :, None], axis=0)
    o2 = jnp.sum(v3[:, :, DH:] * a2[:, :, None], axis=0)

    wm1 = wm1_ref[...]
    out_cat = jnp.concatenate([o1, o2], axis=-1)
    pre = out_cat @ wm1[:D] + h @ wm1[D:] + bm1_ref[...]
    out_ref[...] = jnp.maximum(pre, 0.0) @ wm2_ref[...] + bm2_ref[...]


def _attn_call(h, ngh3, nt_t, ts2, tw2, tb2, Wq, Wk, Wv, Wm1, bm1_2, Wm2, bm2_2):
    tbc = h.shape[0]
    full = lambda shape: pl.BlockSpec(shape, lambda i: tuple(0 for _ in shape))
    return pl.pallas_call(
        _attn_body,
        grid=(tbc // _R,),
        in_specs=[
            pl.BlockSpec((_R, D), lambda i: (i, 0)),
            pl.BlockSpec((K, _R, D), lambda i: (0, i, 0)),
            pl.BlockSpec((K, _R), lambda i: (0, i)),
            pl.BlockSpec((1, _R), lambda i: (0, i)),
            full((1, D)), full((1, D)),
            full((2 * D, D)), full((2 * D, D)), full((2 * D, D)),
            full((2 * D, D)), full((1, D)),
            full((D, D)), full((1, D)),
        ],
        out_specs=pl.BlockSpec((_R, D), lambda i: (i, 0)),
        out_shape=jax.ShapeDtypeStruct((tbc, D), jnp.float32),
    )(h, ngh3, nt_t, ts2, tw2, tb2, Wq, Wk, Wv, Wm1, bm1_2, Wm2, bm2_2)


_NCHUNK = 6


def kernel(node_feat, memory, time_w, time_b, Wq, Wk, Wv, Wm1, bm1, Wm2, bm2,
           source_nodes, destination_nodes, negative_nodes, edge_times,
           ngh_idx, ngh_times):
    nodes = jnp.concatenate(
        [source_nodes, destination_nodes, negative_nodes]).astype(jnp.int32)
    ts3 = jnp.concatenate([edge_times, edge_times, edge_times])

    combined = _combine(node_feat, memory)

    idx_t = ngh_idx.astype(jnp.int32).T
    nt_t = ngh_times.T
    tw2, tb2 = time_w.reshape(1, D), time_b.reshape(1, D)
    bm1_2, bm2_2 = bm1.reshape(1, D), bm2.reshape(1, D)

    tbc = TB // _NCHUNK
    gather = _make_sc_gather(tbc)
    embs = []
    for j in range(_NCHUNK):
        sl = slice(j * tbc, (j + 1) * tbc)
        ngh_flat, h = gather(combined, idx_t[:, sl].reshape(-1), nodes[sl])
        embs.append(_attn_call(
            h, ngh_flat.reshape(K, tbc, D), nt_t[:, sl],
            ts3[sl].reshape(1, tbc), tw2, tb2,
            Wq, Wk, Wv, Wm1, bm1_2, Wm2, bm2_2))
    return jnp.concatenate(embs, axis=0)


# speedup vs baseline: 7.4816x; 1.2764x over previous
import functools

import jax
import jax.numpy as jnp
from jax import lax
from jax.experimental import pallas as pl
from jax.experimental.pallas import tpu as pltpu
from jax.experimental.pallas import tpu_sc as plsc

N = 100000
D = 128
B = 16384
TB = 3 * B
K = 20
H = 2
DH = D // H


_CRB = 1000


def _combine_body(nf_ref, mem_ref, out_ref):
    out_ref[...] = nf_ref[...] + mem_ref[...]


def _combine(node_feat, memory):
    return pl.pallas_call(
        _combine_body,
        grid=(N // _CRB,),
        in_specs=[pl.BlockSpec((_CRB, D), lambda i: (i, 0)),
                  pl.BlockSpec((_CRB, D), lambda i: (i, 0))],
        out_specs=pl.BlockSpec((_CRB, D), lambda i: (i, 0)),
        out_shape=jax.ShapeDtypeStruct((N, D), jnp.float32),
    )(node_feat, memory)


_NC, _NS = 2, 16
_NW = _NC * _NS
_CH = 128


@functools.lru_cache(maxsize=None)
def _make_sc_gather(tbc):
    pwn = tbc * K // _NW
    pwh = tbc // _NW
    assert pwn % (2 * _CH) == 0 and pwh % (2 * _CH) == 0

    def body(table, idx_n, idx_h, out_n, out_h, idxl, rows, sg0, sg1,
             ss0, ss1):
        wid = lax.axis_index("s") * _NC + lax.axis_index("c")
        pltpu.sync_copy(idx_n.at[pl.ds(wid * pwn, pwn)], idxl.at[pl.ds(0, pwn)])
        pltpu.sync_copy(idx_h.at[pl.ds(wid * pwh, pwh)],
                        idxl.at[pl.ds(pwn, pwh)])
        sg = (sg0, sg1)
        ss = (ss0, ss1)

        def run(ibase, out_hbm, obase, n_chunks):
            def g_copy(i, b):
                return pltpu.make_async_copy(
                    table.at[idxl.at[pl.ds(ibase + i * _CH, _CH)]],
                    rows.at[b], sg[b])

            def s_copy(i, b):
                return pltpu.make_async_copy(
                    rows.at[b], out_hbm.at[pl.ds(obase + i * _CH, _CH)], ss[b])

            for b in (0, 1):
                g_copy(b, b).start()

            def loop_body(g, carry):
                for b in (0, 1):
                    i = 2 * g + b
                    g_copy(i, b).wait()
                    s_copy(i, b).start()
                for b in (0, 1):
                    i = 2 * g + b

                    def _prefetch(i=i, b=b):
                        s_copy(i, b).wait()
                        g_copy(i + 2, b).start()

                    pl.when(i + 2 < n_chunks)(_prefetch)
                return carry

            lax.fori_loop(0, n_chunks // 2, loop_body, 0)
            for b in (0, 1):
                s_copy(n_chunks - 2 + b, b).wait()

        run(0, out_n, wid * pwn, pwn // _CH)
        run(pwn, out_h, wid * pwh, pwh // _CH)

    return pl.kernel(
        body,
        out_type=(jax.ShapeDtypeStruct((tbc * K, D), jnp.float32),
                  jax.ShapeDtypeStruct((tbc, D), jnp.float32)),
        mesh=plsc.VectorSubcoreMesh(core_axis_name="c", subcore_axis_name="s"),
        scratch_types=[
            pltpu.VMEM((pwn + pwh,), jnp.int32),
            pltpu.VMEM((2, _CH, D), jnp.float32),
            pltpu.SemaphoreType.DMA,
            pltpu.SemaphoreType.DMA,
            pltpu.SemaphoreType.DMA,
            pltpu.SemaphoreType.DMA,
        ],
    )


_R = 128

_INV2PI = 0.15915494309189535
_COSC = (9.9995902495e-01, -1.9730942534e+01, 6.4671443424e+01,
         -8.2390811065e+01, 4.5621052378e+01)


def _fast_cos(x):
    r = x * _INV2PI
    f = r - jnp.floor(r + 0.5)
    u = f * f
    p = jnp.float32(_COSC[4])
    for c in (_COSC[3], _COSC[2], _COSC[1], _COSC[0]):
        p = p * u + jnp.float32(c)
    return p


def _attn_body(h_ref, ngh_ref, nt_ref, ts_ref, tw_ref, tb_ref, wq_ref,
               wk_ref, wv_ref, wm1_ref, bm1_ref, wm2_ref, bm2_ref, out_ref):
    h = h_ref[...]
    ngh2 = ngh_ref[...].reshape(K * _R, D)
    nt = nt_ref[...]
    ts = ts_ref[...]
    w = tw_ref[...]
    b = tb_ref[...]

    dt = ts - nt
    te = _fast_cos(dt[:, :, None] * w[None, :, :] + b[None, :, :])
    te2 = te.reshape(K * _R, D)

    wk = wk_ref[...]
    wv = wv_ref[...]
    kmat = ngh2 @ wk[:D] + te2 @ wk[D:]
    vmat = ngh2 @ wv[:D] + te2 @ wv[D:]

    wq = wq_ref[...]
    scale = 1.0 / (DH ** 0.5)
    q = (h @ wq[:D] + jnp.cos(b) @ wq[D:]) * scale

    k3 = kmat.reshape(K, _R, D)
    v3 = vmat.reshape(K, _R, D)
    sfull = k3 * q[None, :, :]
    e1 = jnp.exp(jnp.sum(sfull[:, :, :DH], axis=-1))
    e2 = jnp.exp(jnp.sum(sfull[:, :, DH:], axis=-1))
    r1 = jnp.reciprocal(jnp.sum(e1, axis=0))
    r2 = jnp.reciprocal(jnp.sum(e2, axis=0))
    o1 = jnp.sum(v3[:, :, :DH] * e1[:, :, None], axis=0) * r1[:, None]
    o2 = jnp.sum(v3[:, :, DH:] * e2[:, :, None], axis=0) * r2[:, None]

    wm1 = wm1_ref[...]
    out_cat = jnp.concatenate([o1, o2], axis=-1)
    pre = out_cat @ wm1[:D] + h @ wm1[D:] + bm1_ref[...]
    out_ref[...] = jnp.maximum(pre, 0.0) @ wm2_ref[...] + bm2_ref[...]


def _attn_call(h, ngh3, nt_t, ts2, tw2, tb2, Wq, Wk, Wv, Wm1, bm1_2, Wm2, bm2_2):
    tbc = h.shape[0]
    full = lambda shape: pl.BlockSpec(shape, lambda i: tuple(0 for _ in shape))
    return pl.pallas_call(
        _attn_body,
        grid=(tbc // _R,),
        in_specs=[
            pl.BlockSpec((_R, D), lambda i: (i, 0)),
            pl.BlockSpec((K, _R, D), lambda i: (0, i, 0)),
            pl.BlockSpec((K, _R), lambda i: (0, i)),
            pl.BlockSpec((1, _R), lambda i: (0, i)),
            full((1, D)), full((1, D)),
            full((2 * D, D)), full((2 * D, D)), full((2 * D, D)),
            full((2 * D, D)), full((1, D)),
            full((D, D)), full((1, D)),
        ],
        out_specs=pl.BlockSpec((_R, D), lambda i: (i, 0)),
        out_shape=jax.ShapeDtypeStruct((tbc, D), jnp.float32),
    )(h, ngh3, nt_t, ts2, tw2, tb2, Wq, Wk, Wv, Wm1, bm1_2, Wm2, bm2_2)


_NCHUNK = 6


def kernel(node_feat, memory, time_w, time_b, Wq, Wk, Wv, Wm1, bm1, Wm2, bm2,
           source_nodes, destination_nodes, negative_nodes, edge_times,
           ngh_idx, ngh_times):
    nodes = jnp.concatenate(
        [source_nodes, destination_nodes, negative_nodes]).astype(jnp.int32)
    ts3 = jnp.concatenate([edge_times, edge_times, edge_times])

    combined = _combine(node_feat, memory)

    idx_t = ngh_idx.astype(jnp.int32).T
    nt_t = ngh_times.T
    tw2, tb2 = time_w.reshape(1, D), time_b.reshape(1, D)
    bm1_2, bm2_2 = bm1.reshape(1, D), bm2.reshape(1, D)

    tbc = TB // _NCHUNK
    gather = _make_sc_gather(tbc)
    embs = []
    for j in range(_NCHUNK):
        sl = slice(j * tbc, (j + 1) * tbc)
        ngh_flat, h = gather(combined, idx_t[:, sl].reshape(-1), nodes[sl])
        embs.append(_attn_call(
            h, ngh_flat.reshape(K, tbc, D), nt_t[:, sl],
            ts3[sl].reshape(1, tbc), tw2, tb2,
            Wq, Wk, Wv, Wm1, bm1_2, Wm2, bm2_2))
    return jnp.concatenate(embs, axis=0)


# speedup vs baseline: 9.3957x; 1.2558x over previous
import functools

import jax
import jax.numpy as jnp
from jax import lax
from jax.experimental import pallas as pl
from jax.experimental.pallas import tpu as pltpu
from jax.experimental.pallas import tpu_sc as plsc

N = 100000
D = 128
B = 16384
TB = 3 * B
K = 20
H = 2
DH = D // H


_CRB = 1000


def _combine_body(nf_ref, mem_ref, out_ref):
    out_ref[...] = nf_ref[...] + mem_ref[...]


def _combine(node_feat, memory):
    return pl.pallas_call(
        _combine_body,
        grid=(N // _CRB,),
        in_specs=[pl.BlockSpec((_CRB, D), lambda i: (i, 0)),
                  pl.BlockSpec((_CRB, D), lambda i: (i, 0))],
        out_specs=pl.BlockSpec((_CRB, D), lambda i: (i, 0)),
        out_shape=jax.ShapeDtypeStruct((N, D), jnp.float32),
    )(node_feat, memory)


_NC, _NS = 2, 16
_NW = _NC * _NS
_CH = 128


@functools.lru_cache(maxsize=None)
def _make_sc_gather(tbc):
    pwn = tbc * K // _NW
    pwh = tbc // _NW
    assert pwn % (2 * _CH) == 0 and pwh % (2 * _CH) == 0

    def body(table, idx_n, idx_h, out_n, out_h, idxl, rows, sg0, sg1,
             ss0, ss1):
        wid = lax.axis_index("s") * _NC + lax.axis_index("c")
        pltpu.sync_copy(idx_n.at[pl.ds(wid * pwn, pwn)], idxl.at[pl.ds(0, pwn)])
        pltpu.sync_copy(idx_h.at[pl.ds(wid * pwh, pwh)],
                        idxl.at[pl.ds(pwn, pwh)])
        sg = (sg0, sg1)
        ss = (ss0, ss1)

        def run(ibase, out_hbm, obase, n_chunks):
            def g_copy(i, b):
                return pltpu.make_async_copy(
                    table.at[idxl.at[pl.ds(ibase + i * _CH, _CH)]],
                    rows.at[b], sg[b])

            def s_copy(i, b):
                return pltpu.make_async_copy(
                    rows.at[b], out_hbm.at[pl.ds(obase + i * _CH, _CH)], ss[b])

            for b in (0, 1):
                g_copy(b, b).start()

            def loop_body(g, carry):
                for b in (0, 1):
                    i = 2 * g + b
                    g_copy(i, b).wait()
                    s_copy(i, b).start()
                for b in (0, 1):
                    i = 2 * g + b

                    def _prefetch(i=i, b=b):
                        s_copy(i, b).wait()
                        g_copy(i + 2, b).start()

                    pl.when(i + 2 < n_chunks)(_prefetch)
                return carry

            lax.fori_loop(0, n_chunks // 2, loop_body, 0)
            for b in (0, 1):
                s_copy(n_chunks - 2 + b, b).wait()

        run(0, out_n, wid * pwn, pwn // _CH)
        run(pwn, out_h, wid * pwh, pwh // _CH)

    return pl.kernel(
        body,
        out_type=(jax.ShapeDtypeStruct((tbc * K, D), jnp.float32),
                  jax.ShapeDtypeStruct((tbc, D), jnp.float32)),
        mesh=plsc.VectorSubcoreMesh(core_axis_name="c", subcore_axis_name="s"),
        scratch_types=[
            pltpu.VMEM((pwn + pwh,), jnp.int32),
            pltpu.VMEM((2, _CH, D), jnp.float32),
            pltpu.SemaphoreType.DMA,
            pltpu.SemaphoreType.DMA,
            pltpu.SemaphoreType.DMA,
            pltpu.SemaphoreType.DMA,
        ],
    )


_R = 128

_INV2PI = 0.15915494309189535
_COSC = (9.9995902495e-01, -1.9730942534e+01, 6.4671443424e+01,
         -8.2390811065e+01, 4.5621052378e+01)


def _fast_cos_pre(r):
    f = r - jnp.floor(r + 0.5)
    u = f * f
    p = jnp.float32(_COSC[4])
    for c in (_COSC[3], _COSC[2], _COSC[1], _COSC[0]):
        p = p * u + jnp.float32(c)
    return p


def _attn_body(h_ref, ngh_ref, nt_ref, ts_ref, tw_ref, tb_ref, wq_ref,
               wk_ref, wv_ref, wm1_ref, bm1_ref, wm2_ref, bm2_ref, out_ref):
    h = h_ref[...]
    ngh2 = ngh_ref[...].reshape(K * _R, D)
    nt = nt_ref[...]
    ts = ts_ref[...]
    w = tw_ref[...]
    b = tb_ref[...]

    dt = ts - nt
    te = _fast_cos_pre(dt[:, :, None] * w[None, :, :] + b[None, :, :])
    te2 = te.reshape(K * _R, D)

    wk = wk_ref[...]
    wv = wv_ref[...]
    kmat = ngh2 @ wk[:D] + te2 @ wk[D:]
    vmat = ngh2 @ wv[:D] + te2 @ wv[D:]

    wq = wq_ref[...]
    scale = 1.0 / (DH ** 0.5)
    q = (h @ wq[:D] + _fast_cos_pre(b) @ wq[D:]) * scale

    k3 = kmat.reshape(K, _R, D)
    sfull2 = (k3 * q[None, :, :]).reshape(K * _R, D)
    ii = lax.broadcasted_iota(jnp.int32, (D, D), 0)
    jj = lax.broadcasted_iota(jnp.int32, (D, D), 1)
    hsel = ((ii < DH) == (jj < DH)).astype(jnp.float32)
    ebc = jnp.exp(sfull2 @ hsel)
    wei = vmat * ebc
    den = jnp.sum(ebc.reshape(K, _R, D), axis=0)
    o_num = jnp.sum(wei.reshape(K, _R, D), axis=0)
    out_cat = o_num * jnp.reciprocal(den)

    wm1 = wm1_ref[...]
    pre = out_cat @ wm1[:D] + h @ wm1[D:] + bm1_ref[...]
    out_ref[...] = jnp.maximum(pre, 0.0) @ wm2_ref[...] + bm2_ref[...]


def _attn_call(h, ngh3, nt_t, ts2, tw2, tb2, Wq, Wk, Wv, Wm1, bm1_2, Wm2, bm2_2):
    tbc = h.shape[0]
    full = lambda shape: pl.BlockSpec(shape, lambda i: tuple(0 for _ in shape))
    return pl.pallas_call(
        _attn_body,
        grid=(tbc // _R,),
        in_specs=[
            pl.BlockSpec((_R, D), lambda i: (i, 0)),
            pl.BlockSpec((K, _R, D), lambda i: (0, i, 0)),
            pl.BlockSpec((K, _R), lambda i: (0, i)),
            pl.BlockSpec((1, _R), lambda i: (0, i)),
            full((1, D)), full((1, D)),
            full((2 * D, D)), full((2 * D, D)), full((2 * D, D)),
            full((2 * D, D)), full((1, D)),
            full((D, D)), full((1, D)),
        ],
        out_specs=pl.BlockSpec((_R, D), lambda i: (i, 0)),
        out_shape=jax.ShapeDtypeStruct((tbc, D), jnp.float32),
    )(h, ngh3, nt_t, ts2, tw2, tb2, Wq, Wk, Wv, Wm1, bm1_2, Wm2, bm2_2)


_NCHUNK = 6


def kernel(node_feat, memory, time_w, time_b, Wq, Wk, Wv, Wm1, bm1, Wm2, bm2,
           source_nodes, destination_nodes, negative_nodes, edge_times,
           ngh_idx, ngh_times):
    nodes = jnp.concatenate(
        [source_nodes, destination_nodes, negative_nodes]).astype(jnp.int32)
    ts3 = jnp.concatenate([edge_times, edge_times, edge_times])

    combined = _combine(node_feat, memory)

    idx_t = ngh_idx.astype(jnp.int32).T
    nt_t = ngh_times.T
    tw2 = (time_w * _INV2PI).reshape(1, D)
    tb2 = (time_b * _INV2PI).reshape(1, D)
    bm1_2, bm2_2 = bm1.reshape(1, D), bm2.reshape(1, D)

    tbc = TB // _NCHUNK
    gather = _make_sc_gather(tbc)
    embs = []
    for j in range(_NCHUNK):
        sl = slice(j * tbc, (j + 1) * tbc)
        ngh_flat, h = gather(combined, idx_t[:, sl].reshape(-1), nodes[sl])
        embs.append(_attn_call(
            h, ngh_flat.reshape(K, tbc, D), nt_t[:, sl],
            ts3[sl].reshape(1, tbc), tw2, tb2,
            Wq, Wk, Wv, Wm1, bm1_2, Wm2, bm2_2))
    return jnp.concatenate(embs, axis=0)
